# Initial kernel scaffold; baseline (speedup 1.0000x reference)
import jax
import jax.numpy as jnp
from jax.experimental import pallas as pl


def kernel(y_real, y_imag, h_real, h_imag, s_diag):
    raise NotImplementedError("write your pallas kernel here")


# speedup vs baseline: 539.4685x; 539.4685x over previous
import numpy as np
import jax
import jax.numpy as jnp
from jax.experimental import pallas as pl
from jax.experimental.pallas import tpu as pltpu

_INV10 = float(1.0 / np.sqrt(10.0))
_PAMS = [-3.0 * _INV10, -1.0 * _INV10, 1.0 * _INV10, 3.0 * _INV10]
_CLIP = 20.0
_BIG = 1e9

_M, _S, _S2 = 8, 4, 8
_SUB, _LANE = 8, 128
_BB = _SUB * _LANE


def _ce(a, b):
    da, ka = a
    db, kb = b
    m = da <= db
    lo = (jnp.where(m, da, db), jnp.where(m, ka, kb))
    hi = (jnp.where(m, db, da), jnp.where(m, kb, ka))
    return lo, hi


def _lo(a, b):
    da, ka = a
    db, kb = b
    m = da <= db
    return (jnp.where(m, da, db), jnp.where(m, ka, kb))


def _sort4(r):
    r = list(r)
    for (i, j) in ((0, 1), (2, 3), (0, 2), (1, 3), (1, 2)):
        r[i], r[j] = _ce(r[i], r[j])
    return r


def _bitonic_merge(seq):
    n = len(seq)
    if n == 1:
        return seq
    half = n // 2
    lo, hi = [], []
    for i in range(half):
        l, h = _ce(seq[i], seq[i + half])
        lo.append(l)
        hi.append(h)
    return _bitonic_merge(lo) + _bitonic_merge(hi)


def _merge(a, b):
    return _bitonic_merge(list(a) + list(reversed(b)))


def _select16(runs):
    r8 = [_merge(runs[2 * i], runs[2 * i + 1]) for i in range(8)]
    r16 = [_merge(r8[2 * i], r8[2 * i + 1]) for i in range(4)]
    a, b, c, d = r16
    e = [_lo(a[i], b[15 - i]) for i in range(16)]
    f = [_lo(c[i], d[15 - i]) for i in range(16)]
    e = _bitonic_merge(e)
    f = _bitonic_merge(f)
    return [_lo(e[i], f[15 - i]) for i in range(16)]


def _body(hr_ref, hi_ref, yr_ref, yi_ref, s_ref, out_ref):
    f32 = jnp.float32
    hr = [[hr_ref[m, s] for s in range(_S)] for m in range(_M)]
    hi = [[hi_ref[m, s] for s in range(_S)] for m in range(_M)]
    yr = [yr_ref[m] for m in range(_M)]
    yi = [yi_ref[m] for m in range(_M)]
    w2 = [2.0 / s_ref[m] for m in range(_M)]

    hrw = [[hr[m][s] * w2[m] for s in range(_S)] for m in range(_M)]
    hiw = [[hi[m][s] * w2[m] for s in range(_S)] for m in range(_M)]

    P = [[None] * _S for _ in range(_S)]
    normA = [None] * _S
    for s in range(_S):
        for t in range(s, _S):
            accA = hrw[0][s] * hr[0][t]
            accC = hiw[0][s] * hi[0][t]
            for m in range(1, _M):
                accA = accA + hrw[m][s] * hr[m][t]
                accC = accC + hiw[m][s] * hi[m][t]
            P[s][t] = accA + accC
            P[t][s] = P[s][t]
            if s == t:
                normA[s] = P[s][s]
    X = [[None] * _S for _ in range(_S)]
    for s in range(_S):
        for t in range(_S):
            if s == t:
                continue
            if s < t:
                acc = hiw[0][s] * hr[0][t] - hrw[0][s] * hi[0][t]
                for m in range(1, _M):
                    acc = acc + (hiw[m][s] * hr[m][t] - hrw[m][s] * hi[m][t])
                X[s][t] = acc
            else:
                X[s][t] = -X[t][s]

    br = [None] * _S
    bi = [None] * _S
    for s in range(_S):
        a = hrw[0][s] * yr[0] + hiw[0][s] * yi[0]
        b = hrw[0][s] * yi[0] - hiw[0][s] * yr[0]
        for m in range(1, _M):
            a = a + hrw[m][s] * yr[m] + hiw[m][s] * yi[m]
            b = b + hrw[m][s] * yi[m] - hiw[m][s] * yr[m]
        br[s] = a
        bi[s] = b

    Gf = [[None] * _S2 for _ in range(_S2)]
    for s in range(_S):
        for t in range(_S):
            Gf[s][t] = P[s][t]
            Gf[4 + s][4 + t] = P[s][t]
            Gf[s][4 + t] = X[s][t] if s != t else None
            Gf[4 + s][t] = -X[s][t] if s != t else None
    zero = jnp.zeros((_SUB, _LANE), f32)
    for s in range(_S):
        Gf[s][4 + s] = zero
        Gf[4 + s][s] = zero
    bf = br + bi

    n8 = normA + normA
    izero = jnp.zeros((_SUB, _LANE), jnp.int32)
    r8 = []
    for c in range(_S2):
        r = izero
        for t in range(_S2):
            if t == c:
                continue
            gt = n8[t] > n8[c]
            tie = (n8[t] == n8[c]) & (t < c)
            r = r + (gt | tie).astype(jnp.int32)
        r8.append(r)
    oh = [[(r8[c] == i).astype(f32) for c in range(_S2)] for i in range(_S2)]

    tmp = [[None] * _S2 for _ in range(_S2)]
    for i in range(_S2):
        for t in range(_S2):
            acc = oh[i][0] * Gf[0][t]
            for s in range(1, _S2):
                acc = acc + oh[i][s] * Gf[s][t]
            tmp[i][t] = acc
    Gp = [[None] * _S2 for _ in range(_S2)]
    for i in range(_S2):
        for j in range(i + 1):
            acc = oh[j][0] * tmp[i][0]
            for t in range(1, _S2):
                acc = acc + oh[j][t] * tmp[i][t]
            Gp[i][j] = acc
    bp = []
    for i in range(_S2):
        acc = oh[i][0] * bf[0]
        for s in range(1, _S2):
            acc = acc + oh[i][s] * bf[s]
        bp.append(acc)

    Lm = [[None] * _S2 for _ in range(_S2)]
    Dv = [None] * _S2
    for j in range(_S2):
        acc = Gp[j][j]
        for k in range(j):
            acc = acc - Lm[j][k] * Lm[j][k] * Dv[k]
        Dv[j] = acc
        inv = 1.0 / acc
        for i in range(j + 1, _S2):
            a2 = Gp[i][j]
            for k in range(j):
                a2 = a2 - Lm[i][k] * Lm[j][k] * Dv[k]
            Lm[i][j] = a2 * inv
    cv = [None] * _S2
    for i in range(_S2):
        acc = bp[i]
        for j in range(i):
            acc = acc - Lm[i][j] * cv[j]
        cv[i] = acc
    ev = [cv[i] / Dv[i] for i in range(_S2)]

    cand = []
    for i7 in range(4):
        t7 = Dv[7] * (ev[7] - _PAMS[i7]) ** 2
        w6 = ev[6] - Lm[7][6] * _PAMS[i7]
        for i6 in range(4):
            d = t7 + Dv[6] * (w6 - _PAMS[i6]) ** 2
            k = jnp.full((_SUB, _LANE), (i7 << 2) | i6, jnp.int32)
            cand.append((d, k))

    for l in range(5, -1, -1):
        runs = []
        for (d, k) in cand:
            w = ev[l]
            for j in range(l + 1, _S2):
                t = (k >> (2 * (j - l - 1))) & 3
                sym = t.astype(f32) * (2.0 * _INV10) - (3.0 * _INV10)
                w = w - Lm[j][l] * sym
            run = []
            for p in range(4):
                dd = d + Dv[l] * (w - _PAMS[p]) ** 2
                kk = (k << 2) | p
                run.append((dd, kk))
            runs.append(_sort4(run))
        cand = _select16(runs)

    dh = [0.5 * d for (d, _) in cand]
    big = jnp.full((_SUB, _LANE), _BIG, f32)
    for s in range(_S):
        shr = 2 * r8[s]
        shi = 2 * r8[4 + s]
        bits = []
        for (_, k) in cand:
            tr = (k >> shr) & 3
            ti = (k >> shi) & 3
            bits.append((tr >= 2, (tr == 1) | (tr == 2),
                         ti >= 2, (ti == 1) | (ti == 2)))
        for bpos in range(4):
            d0 = big
            d1 = big
            for ci in range(16):
                bb = bits[ci][bpos]
                d0 = jnp.minimum(d0, jnp.where(bb, big, dh[ci]))
                d1 = jnp.minimum(d1, jnp.where(bb, dh[ci], big))
            out_ref[s * 4 + bpos] = jnp.clip(d0 - d1, -_CLIP, _CLIP)


def kernel(y_real, y_imag, h_real, h_imag, s_diag):
    B = y_real.shape[0]
    nblk = B // _BB
    grid = (nblk,)
    hrt = h_real.transpose(1, 2, 0).reshape(_M, _S, nblk * _SUB, _LANE)
    hit = h_imag.transpose(1, 2, 0).reshape(_M, _S, nblk * _SUB, _LANE)
    yrt = y_real.transpose(1, 0).reshape(_M, nblk * _SUB, _LANE)
    yit = y_imag.transpose(1, 0).reshape(_M, nblk * _SUB, _LANE)
    st = s_diag.transpose(1, 0).reshape(_M, nblk * _SUB, _LANE)
    out = pl.pallas_call(
        _body,
        grid=grid,
        in_specs=[
            pl.BlockSpec((_M, _S, _SUB, _LANE), lambda i: (0, 0, i, 0)),
            pl.BlockSpec((_M, _S, _SUB, _LANE), lambda i: (0, 0, i, 0)),
            pl.BlockSpec((_M, _SUB, _LANE), lambda i: (0, i, 0)),
            pl.BlockSpec((_M, _SUB, _LANE), lambda i: (0, i, 0)),
            pl.BlockSpec((_M, _SUB, _LANE), lambda i: (0, i, 0)),
        ],
        out_specs=pl.BlockSpec((16, _SUB, _LANE), lambda i: (0, i, 0)),
        out_shape=jax.ShapeDtypeStruct((16, nblk * _SUB, _LANE), jnp.float32),
    )(hrt, hit, yrt, yit, st)
    return out.reshape(16, B).transpose(1, 0).reshape(B, _S, 4)


# speedup vs baseline: 539.6527x; 1.0003x over previous
import numpy as np
import jax
import jax.numpy as jnp
from jax.experimental import pallas as pl

_INV10 = float(1.0 / np.sqrt(10.0))
_PAMS = [-3.0 * _INV10, -1.0 * _INV10, 1.0 * _INV10, 3.0 * _INV10]
_CLIP = 20.0
_BIG = 1e9

_M, _S, _S2 = 8, 4, 8
_SUB, _LANE = 8, 128
_BB = _SUB * _LANE


def _ce(a, b):
    da, ka = a
    db, kb = b
    m = da <= db
    lo = (jnp.where(m, da, db), jnp.where(m, ka, kb))
    hi = (jnp.where(m, db, da), jnp.where(m, kb, ka))
    return lo, hi


def _lo(a, b):
    da, ka = a
    db, kb = b
    m = da <= db
    return (jnp.where(m, da, db), jnp.where(m, ka, kb))


def _sort4(r):
    r = list(r)
    for (i, j) in ((0, 1), (2, 3), (0, 2), (1, 3), (1, 2)):
        r[i], r[j] = _ce(r[i], r[j])
    return r


def _bitonic_merge(seq):
    n = len(seq)
    if n == 1:
        return seq
    half = n // 2
    lo, hi = [], []
    for i in range(half):
        l, h = _ce(seq[i], seq[i + half])
        lo.append(l)
        hi.append(h)
    return _bitonic_merge(lo) + _bitonic_merge(hi)


def _merge(a, b):
    return _bitonic_merge(list(a) + list(reversed(b)))


def _select16(runs):
    r8 = [_merge(runs[2 * i], runs[2 * i + 1]) for i in range(8)]
    r16 = [_merge(r8[2 * i], r8[2 * i + 1]) for i in range(4)]
    a, b, c, d = r16
    e = [_lo(a[i], b[15 - i]) for i in range(16)]
    f = [_lo(c[i], d[15 - i]) for i in range(16)]
    e = _bitonic_merge(e)
    f = _bitonic_merge(f)
    return [_lo(e[i], f[15 - i]) for i in range(16)]


def _body(hr_ref, hi_ref, yr_ref, yi_ref, s_ref, out_ref):
    f32 = jnp.float32
    hr = [[hr_ref[m, s] for s in range(_S)] for m in range(_M)]
    hi = [[hi_ref[m, s] for s in range(_S)] for m in range(_M)]
    yr = [yr_ref[m] for m in range(_M)]
    yi = [yi_ref[m] for m in range(_M)]
    w2 = [2.0 / s_ref[m] for m in range(_M)]

    hrw = [[hr[m][s] * w2[m] for s in range(_S)] for m in range(_M)]
    hiw = [[hi[m][s] * w2[m] for s in range(_S)] for m in range(_M)]

    P = [[None] * _S for _ in range(_S)]
    normA = [None] * _S
    for s in range(_S):
        for t in range(s, _S):
            accA = hrw[0][s] * hr[0][t]
            accC = hiw[0][s] * hi[0][t]
            for m in range(1, _M):
                accA = accA + hrw[m][s] * hr[m][t]
                accC = accC + hiw[m][s] * hi[m][t]
            P[s][t] = accA + accC
            P[t][s] = P[s][t]
            if s == t:
                normA[s] = P[s][s]
    X = [[None] * _S for _ in range(_S)]
    for s in range(_S):
        for t in range(_S):
            if s == t:
                continue
            if s < t:
                acc = hiw[0][s] * hr[0][t] - hrw[0][s] * hi[0][t]
                for m in range(1, _M):
                    acc = acc + (hiw[m][s] * hr[m][t] - hrw[m][s] * hi[m][t])
                X[s][t] = acc
            else:
                X[s][t] = -X[t][s]

    br = [None] * _S
    bi = [None] * _S
    for s in range(_S):
        a = hrw[0][s] * yr[0] + hiw[0][s] * yi[0]
        b = hrw[0][s] * yi[0] - hiw[0][s] * yr[0]
        for m in range(1, _M):
            a = a + hrw[m][s] * yr[m] + hiw[m][s] * yi[m]
            b = b + hrw[m][s] * yi[m] - hiw[m][s] * yr[m]
        br[s] = a
        bi[s] = b

    Gf = [[None] * _S2 for _ in range(_S2)]
    for s in range(_S):
        for t in range(_S):
            Gf[s][t] = P[s][t]
            Gf[4 + s][4 + t] = P[s][t]
            Gf[s][4 + t] = X[s][t] if s != t else None
            Gf[4 + s][t] = -X[s][t] if s != t else None
    zero = jnp.zeros((_SUB, _LANE), f32)
    for s in range(_S):
        Gf[s][4 + s] = zero
        Gf[4 + s][s] = zero
    bf = br + bi

    n8 = normA + normA
    izero = jnp.zeros((_SUB, _LANE), jnp.int32)
    r8 = []
    for c in range(_S2):
        r = izero
        for t in range(_S2):
            if t == c:
                continue
            gt = n8[t] > n8[c]
            tie = (n8[t] == n8[c]) & (t < c)
            r = r + (gt | tie).astype(jnp.int32)
        r8.append(r)
    oh = [[(r8[c] == i).astype(f32) for c in range(_S2)] for i in range(_S2)]

    tmp = [[None] * _S2 for _ in range(_S2)]
    for i in range(_S2):
        for t in range(_S2):
            acc = oh[i][0] * Gf[0][t]
            for s in range(1, _S2):
                acc = acc + oh[i][s] * Gf[s][t]
            tmp[i][t] = acc
    Gp = [[None] * _S2 for _ in range(_S2)]
    for i in range(_S2):
        for j in range(i + 1):
            acc = oh[j][0] * tmp[i][0]
            for t in range(1, _S2):
                acc = acc + oh[j][t] * tmp[i][t]
            Gp[i][j] = acc
    bp = []
    for i in range(_S2):
        acc = oh[i][0] * bf[0]
        for s in range(1, _S2):
            acc = acc + oh[i][s] * bf[s]
        bp.append(acc)

    Lm = [[None] * _S2 for _ in range(_S2)]
    Dv = [None] * _S2
    for j in range(_S2):
        acc = Gp[j][j]
        for k in range(j):
            acc = acc - Lm[j][k] * Lm[j][k] * Dv[k]
        Dv[j] = acc
        inv = 1.0 / acc
        for i in range(j + 1, _S2):
            a2 = Gp[i][j]
            for k in range(j):
                a2 = a2 - Lm[i][k] * Lm[j][k] * Dv[k]
            Lm[i][j] = a2 * inv
    cv = [None] * _S2
    for i in range(_S2):
        acc = bp[i]
        for j in range(i):
            acc = acc - Lm[i][j] * cv[j]
        cv[i] = acc
    ev = [cv[i] / Dv[i] for i in range(_S2)]

    cand = []
    for i7 in range(4):
        t7 = Dv[7] * (ev[7] - _PAMS[i7]) ** 2
        w6 = ev[6] - Lm[7][6] * _PAMS[i7]
        for i6 in range(4):
            d = t7 + Dv[6] * (w6 - _PAMS[i6]) ** 2
            k = jnp.full((_SUB, _LANE), (i7 << 2) | i6, jnp.int32)
            cand.append((d, k))

    for l in range(5, -1, -1):
        runs = []
        for (d, k) in cand:
            w = ev[l]
            for j in range(l + 1, _S2):
                t = (k >> (2 * (j - l - 1))) & 3
                sym = t.astype(f32) * (2.0 * _INV10) - (3.0 * _INV10)
                w = w - Lm[j][l] * sym
            run = []
            for p in range(4):
                dd = d + Dv[l] * (w - _PAMS[p]) ** 2
                kk = (k << 2) | p
                run.append((dd, kk))
            runs.append(_sort4(run))
        cand = _select16(runs)

    dh = [0.5 * d for (d, _) in cand]
    big = jnp.full((_SUB, _LANE), _BIG, f32)
    for s in range(_S):
        shr = 2 * r8[s]
        shi = 2 * r8[4 + s]
        bits = []
        for (_, k) in cand:
            tr = (k >> shr) & 3
            ti = (k >> shi) & 3
            bits.append((tr >= 2, (tr == 1) | (tr == 2),
                         ti >= 2, (ti == 1) | (ti == 2)))
        for bpos in range(4):
            d0 = big
            d1 = big
            for ci in range(16):
                bb = bits[ci][bpos]
                d0 = jnp.minimum(d0, jnp.where(bb, big, dh[ci]))
                d1 = jnp.minimum(d1, jnp.where(bb, dh[ci], big))
            out_ref[s * 4 + bpos] = jnp.clip(d0 - d1, -_CLIP, _CLIP)


def kernel(y_real, y_imag, h_real, h_imag, s_diag):
    B = y_real.shape[0]
    nblk = B // _BB
    grid = (nblk,)
    hrt = h_real.transpose(1, 2, 0).reshape(_M, _S, nblk * _SUB, _LANE)
    hit = h_imag.transpose(1, 2, 0).reshape(_M, _S, nblk * _SUB, _LANE)
    yrt = y_real.transpose(1, 0).reshape(_M, nblk * _SUB, _LANE)
    yit = y_imag.transpose(1, 0).reshape(_M, nblk * _SUB, _LANE)
    st = s_diag.transpose(1, 0).reshape(_M, nblk * _SUB, _LANE)
    out = pl.pallas_call(
        _body,
        grid=grid,
        in_specs=[
            pl.BlockSpec((_M, _S, _SUB, _LANE), lambda i: (0, 0, i, 0)),
            pl.BlockSpec((_M, _S, _SUB, _LANE), lambda i: (0, 0, i, 0)),
            pl.BlockSpec((_M, _SUB, _LANE), lambda i: (0, i, 0)),
            pl.BlockSpec((_M, _SUB, _LANE), lambda i: (0, i, 0)),
            pl.BlockSpec((_M, _SUB, _LANE), lambda i: (0, i, 0)),
        ],
        out_specs=pl.BlockSpec((16, _SUB, _LANE), lambda i: (0, i, 0)),
        out_shape=jax.ShapeDtypeStruct((16, nblk * _SUB, _LANE), jnp.float32),
    )(hrt, hit, yrt, yit, st)
    return out.reshape(16, B).transpose(1, 0).reshape(B, _S, 4)
